# Initial kernel scaffold; baseline (speedup 1.0000x reference)
import jax
import jax.numpy as jnp
from jax.experimental import pallas as pl


def kernel(x):
    raise NotImplementedError("write your pallas kernel here")


# speedup vs baseline: 1.3845x; 1.3845x over previous
import jax
import jax.numpy as jnp
from jax.experimental import pallas as pl
from jax.experimental.pallas import tpu as pltpu

_N_TOK = 576
_N_SEL = 86
_STEP = 6
_M = 29
_NEG = -1e30


def _body(x_ref, out_ref, g_ref):
    xb = x_ref[0]
    tok = xb[1:1 + _N_TOK]

    norm = jnp.sqrt(jnp.sum(tok * tok, axis=1, keepdims=True))
    tok_n = tok / norm

    grp = tok_n[: _N_SEL * _STEP].reshape(_N_SEL, _STEP, 768)
    seeds_n = grp[:, 0, :]

    scores = jax.lax.dot_general(
        seeds_n, tok_n, (((1,), (1,)), ((), ())),
        preferred_element_type=jnp.float32)
    nm = jnp.max(scores, axis=0, keepdims=True)

    iota = jax.lax.broadcasted_iota(jnp.int32, (1, _N_TOK), 1)
    seed_mask = (iota % _STEP == 0) & (iota < _N_SEL * _STEP - _STEP + 1)
    nm = jnp.where(seed_mask, _NEG, nm)

    out_ref[0, 0:1, :] = xb[0:1, :]
    out_ref[0, 1:1 + _N_SEL, :] = (
        tok[: _N_SEL * _STEP].reshape(_N_SEL, _STEP, 768)[:, 0, :])

    def extract(j, nm, out_base, save_norm):
        v = jnp.max(nm)
        cand = jnp.where(nm == v, iota, _N_TOK)
        i = jnp.min(cand)
        row = x_ref[0, pl.ds(i + 1, 1), :]
        out_ref[0, pl.ds(out_base + (_M - 1) - j, 1), :] = row
        if save_norm:
            rn = row / jnp.sqrt(jnp.sum(row * row))
            g_ref[pl.ds((_M - 1) - j, 1), :] = rn
        return jnp.where(iota == i, _NEG, nm)

    nm = jax.lax.fori_loop(
        0, _M, lambda j, a: extract(j, a, 1 + _N_SEL, True), nm)

    scores2 = jax.lax.dot_general(
        g_ref[...], tok_n, (((1,), (1,)), ((), ())),
        preferred_element_type=jnp.float32)
    inc = jnp.max(scores2, axis=0, keepdims=True)
    nm = jnp.where(nm == _NEG, _NEG, jnp.maximum(nm, inc))

    jax.lax.fori_loop(
        0, _M, lambda j, a: extract(j, a, 1 + _N_SEL + _M, False), nm)


def kernel(x):
    B = x.shape[0]
    out_tokens = 1 + _N_SEL + 2 * _M
    return pl.pallas_call(
        _body,
        grid=(B,),
        in_specs=[pl.BlockSpec((1, x.shape[1], x.shape[2]),
                               lambda b: (b, 0, 0))],
        out_specs=pl.BlockSpec((1, out_tokens, x.shape[2]),
                               lambda b: (b, 0, 0)),
        out_shape=jax.ShapeDtypeStruct((B, out_tokens, x.shape[2]),
                                       jnp.float32),
        scratch_shapes=[pltpu.VMEM((_M, 768), jnp.float32)],
    )(x)
